# baseline (speedup 1.0000x reference)
import functools

import jax
import jax.numpy as jnp
from jax import lax
from jax.experimental import pallas as pl
from jax.experimental.pallas import tpu as pltpu
from jax.experimental.pallas import tpu_sc as plsc

_N = 10000
_E = 320000
_D = 128
_NC = 2
_NS = 16
_NW = _NC * _NS
_L = 16

_BLK = 12800
_ESC = 12 * _BLK
_ETC = _E - _ESC
_CH = _ESC // _NW
_G = 160
_NG = _CH // _G
_CHT = _ETC // _NW


def _gates_body(nf_ref, w2_ref, b2_ref, out_ref):
    out_ref[...] = lax.dot_general(
        w2_ref[...], nf_ref[...],
        (((1,), (1,)), ((), ())),
        preferred_element_type=jnp.float32,
    ) + b2_ref[...]


def _edge_body(ef_ref, we_ref, c_ref):
    c_ref[...] = lax.dot_general(
        ef_ref[...], we_ref[...],
        (((1,), (1,)), ((), ())),
        preferred_element_type=jnp.float32,
    )


def _sigmoid(m):
    return 1.0 / (1.0 + jnp.exp(-m))


def _sc_heavy_body(tab_hbm, src_hbm, dst_hbm, ef_hbm, w_hbm, y_hbm,
                   tab_v, src_v, dst_v, w_v, y_v, tile0, tile1, sem0, sem1):
    cid = lax.axis_index("c")
    sid = lax.axis_index("s")
    wid = sid * _NC + cid
    ebase = wid * _CH
    pltpu.sync_copy(tab_hbm, tab_v)
    pltpu.sync_copy(src_hbm.at[pl.ds(ebase, _CH)], src_v)
    pltpu.sync_copy(dst_hbm.at[pl.ds(ebase, _CH)], dst_v)
    pltpu.sync_copy(w_hbm, w_v)

    iota128 = lax.iota(jnp.int32, _L) * _D
    wvecs = [w_v[pl.ds(cch * _L, _L)] for cch in range(_D // _L)]

    def start(g, tile, sem):
        pltpu.make_async_copy(
            ef_hbm.at[pl.ds((ebase + g * _G) * _D, _G * _D)], tile, sem
        ).start()

    def wait(g, tile, sem):
        pltpu.make_async_copy(
            ef_hbm.at[pl.ds((ebase + g * _G) * _D, _G * _D)], tile, sem
        ).wait()

    def compute(g, tile):
        def jbody(j, carry):
            base_j = j * (_L * _D)
            acc = jnp.zeros((_L,), jnp.float32)
            for k in range(_D):
                idx = iota128 + (base_j + k)
                acc = acc + plsc.load_gather(tile, [idx]) * wvecs[k // _L][k % _L]
            e_off = g * _G + j * _L
            si = src_v[pl.ds(e_off, _L)]
            di = dst_v[pl.ds(e_off, _L)] + _N
            m = acc + plsc.load_gather(tab_v, [si]) + plsc.load_gather(tab_v, [di])
            y_v[pl.ds(e_off, _L)] = _sigmoid(m)
            return carry

        lax.fori_loop(0, _G // _L, jbody, 0)

    start(0, tile0, sem0)

    def body2(t, carry):
        g0 = 2 * t
        wait(g0, tile0, sem0)
        start(g0 + 1, tile1, sem1)
        compute(g0, tile0)
        wait(g0 + 1, tile1, sem1)

        @pl.when(t < _NG // 2 - 1)
        def _():
            start(g0 + 2, tile0, sem0)

        compute(g0 + 1, tile1)
        return carry

    lax.fori_loop(0, _NG // 2, body2, 0)
    pltpu.sync_copy(y_v, y_hbm.at[pl.ds(ebase, _CH)])


def _sc_final_body(tab_hbm, src_hbm, dst_hbm, c_hbm, y_hbm,
                   tab_v, src_v, dst_v, c_v, y_v):
    cid = lax.axis_index("c")
    sid = lax.axis_index("s")
    wid = sid * _NC + cid
    base = _ESC + wid * _CHT
    pltpu.sync_copy(tab_hbm, tab_v)
    pltpu.sync_copy(src_hbm.at[pl.ds(base, _CHT)], src_v)
    pltpu.sync_copy(dst_hbm.at[pl.ds(base, _CHT)], dst_v)
    pltpu.sync_copy(c_hbm.at[pl.ds(wid * _CHT, _CHT)], c_v)

    def body(i, carry):
        off = i * _L
        si = src_v[pl.ds(off, _L)]
        di = dst_v[pl.ds(off, _L)] + _N
        m = (plsc.load_gather(tab_v, [si]) + plsc.load_gather(tab_v, [di])
             + c_v[pl.ds(off, _L)])
        y_v[pl.ds(off, _L)] = _sigmoid(m)
        return carry

    lax.fori_loop(0, _CHT // _L, body, 0)
    pltpu.sync_copy(y_v, y_hbm.at[pl.ds(wid * _CHT, _CHT)])


@jax.jit
def kernel(node_feats, edge_feats, edge_index, W_src, b_src, W_dst, b_dst,
           W_edge, b_edge):
    src = edge_index[0].astype(jnp.int32)
    dst = edge_index[1].astype(jnp.int32)
    w2 = jnp.concatenate([W_src, W_dst], axis=0)
    b2 = jnp.stack([b_src, b_dst + b_edge]).reshape(2, 1)

    gates = pl.pallas_call(
        _gates_body,
        out_shape=jax.ShapeDtypeStruct((2, _N), jnp.float32),
    )(node_feats, w2, b2)
    table = gates.reshape(2 * _N)

    mesh = plsc.VectorSubcoreMesh(core_axis_name="c", subcore_axis_name="s")
    sc_params = pltpu.CompilerParams(needs_layout_passes=False)

    sc_heavy = pl.kernel(
        _sc_heavy_body,
        out_type=jax.ShapeDtypeStruct((_ESC,), jnp.float32),
        mesh=mesh,
        compiler_params=sc_params,
        scratch_types=[
            pltpu.VMEM((2 * _N,), jnp.float32),
            pltpu.VMEM((_CH,), jnp.int32),
            pltpu.VMEM((_CH,), jnp.int32),
            pltpu.VMEM((_D,), jnp.float32),
            pltpu.VMEM((_CH,), jnp.float32),
            pltpu.VMEM((_G * _D,), jnp.float32),
            pltpu.VMEM((_G * _D,), jnp.float32),
            pltpu.SemaphoreType.DMA,
            pltpu.SemaphoreType.DMA,
        ],
    )
    y_sc = sc_heavy(table, src, dst, edge_feats.reshape(_E * _D),
                    W_edge.reshape(_D))

    c = pl.pallas_call(
        _edge_body,
        grid=(_ETC // _BLK,),
        in_specs=[
            pl.BlockSpec((_BLK, _D), lambda i: (i + _ESC // _BLK, 0)),
            pl.BlockSpec((1, _D), lambda i: (0, 0)),
        ],
        out_specs=pl.BlockSpec((_BLK, 1), lambda i: (i, 0)),
        out_shape=jax.ShapeDtypeStruct((_ETC, 1), jnp.float32),
    )(edge_feats, W_edge)

    sc_final = pl.kernel(
        _sc_final_body,
        out_type=jax.ShapeDtypeStruct((_ETC,), jnp.float32),
        mesh=mesh,
        compiler_params=sc_params,
        scratch_types=[
            pltpu.VMEM((2 * _N,), jnp.float32),
            pltpu.VMEM((_CHT,), jnp.int32),
            pltpu.VMEM((_CHT,), jnp.int32),
            pltpu.VMEM((_CHT,), jnp.float32),
            pltpu.VMEM((_CHT,), jnp.float32),
        ],
    )
    y_tc = sc_final(table, src, dst, c.reshape(_ETC))

    return jnp.concatenate([y_sc, y_tc]).reshape(_E, 1)


# speedup vs baseline: 2.4084x; 2.4084x over previous
import functools

import jax
import jax.numpy as jnp
from jax import lax
from jax.experimental import pallas as pl
from jax.experimental.pallas import tpu as pltpu
from jax.experimental.pallas import tpu_sc as plsc

_N = 10000
_E = 320000
_D = 128
_NC = 2
_NS = 16
_NW = _NC * _NS
_L = 16

_BLK = 12800
_ESC = 12 * _BLK
_ETC = _E - _ESC
_CH = _ESC // _NW
_G = 160
_NG = _CH // _G
_CHT = _ETC // _NW


def _gates_body(nf_ref, w2_ref, b2_ref, out_ref):
    out_ref[...] = lax.dot_general(
        w2_ref[...], nf_ref[...],
        (((1,), (1,)), ((), ())),
        preferred_element_type=jnp.float32,
    ) + b2_ref[...]


def _edge_body(ef_ref, we_ref, c_ref):
    c_ref[...] = lax.dot_general(
        ef_ref[...], we_ref[...],
        (((1,), (1,)), ((), ())),
        preferred_element_type=jnp.float32,
    )


def _sigmoid(m):
    return 1.0 / (1.0 + jnp.exp(-m))


def _vperm(v, p):
    return lax.gather(
        v, p.reshape(_L, 1),
        lax.GatherDimensionNumbers(
            offset_dims=(), collapsed_slice_dims=(0,), start_index_map=(0,)),
        slice_sizes=(1,),
        mode=lax.GatherScatterMode.PROMISE_IN_BOUNDS)


def _sc_heavy_body(tab_hbm, src_hbm, dst_hbm, ef_hbm, w_hbm, y_hbm,
                   tab_v, src_v, dst_v, w_v, y_v, tile0, tile1, dot_v,
                   sem0, sem1):
    cid = lax.axis_index("c")
    sid = lax.axis_index("s")
    wid = sid * _NC + cid
    ebase = wid * _CH
    pltpu.sync_copy(tab_hbm, tab_v)
    pltpu.sync_copy(src_hbm.at[pl.ds(ebase, _CH)], src_v)
    pltpu.sync_copy(dst_hbm.at[pl.ds(ebase, _CH)], dst_v)
    pltpu.sync_copy(w_hbm, w_v)

    wvecs = [w_v[pl.ds(cch * _L, _L)] for cch in range(_D // _L)]
    lane = lax.iota(jnp.int32, _L)
    perms = [lane ^ (1 << s) for s in range(4)]
    onehots = [lane == e for e in range(_L)]

    def start(g, tile, sem):
        pltpu.make_async_copy(
            ef_hbm.at[pl.ds((ebase + g * _G) * _D, _G * _D)], tile, sem
        ).start()

    def wait(g, tile, sem):
        pltpu.make_async_copy(
            ef_hbm.at[pl.ds((ebase + g * _G) * _D, _G * _D)], tile, sem
        ).wait()

    def compute(g, tile):
        def jbody(j, carry):
            base_j = j * (_L * _D)
            for e in range(_L):
                row = base_j + e * _D
                parts = [tile[pl.ds(row + cch * _L, _L)] * wvecs[cch]
                         for cch in range(_D // _L)]
                while len(parts) > 1:
                    parts = [parts[i] + parts[i + 1]
                             for i in range(0, len(parts), 2)]
                v = parts[0]
                for p in perms:
                    v = v + _vperm(v, p)
                plsc.store_scatter(dot_v, [lane], v, mask=onehots[e])
            e_off = g * _G + j * _L
            si = src_v[pl.ds(e_off, _L)]
            di = dst_v[pl.ds(e_off, _L)] + _N
            m = (dot_v[pl.ds(0, _L)] + plsc.load_gather(tab_v, [si])
                 + plsc.load_gather(tab_v, [di]))
            y_v[pl.ds(e_off, _L)] = _sigmoid(m)
            return carry

        lax.fori_loop(0, _G // _L, jbody, 0)

    start(0, tile0, sem0)

    def body2(t, carry):
        g0 = 2 * t
        wait(g0, tile0, sem0)
        start(g0 + 1, tile1, sem1)
        compute(g0, tile0)
        wait(g0 + 1, tile1, sem1)

        @pl.when(t < _NG // 2 - 1)
        def _():
            start(g0 + 2, tile0, sem0)

        compute(g0 + 1, tile1)
        return carry

    lax.fori_loop(0, _NG // 2, body2, 0)
    pltpu.sync_copy(y_v, y_hbm.at[pl.ds(ebase, _CH)])


def _sc_final_body(tab_hbm, src_hbm, dst_hbm, c_hbm, y_hbm,
                   tab_v, src_v, dst_v, c_v, y_v):
    cid = lax.axis_index("c")
    sid = lax.axis_index("s")
    wid = sid * _NC + cid
    base = _ESC + wid * _CHT
    pltpu.sync_copy(tab_hbm, tab_v)
    pltpu.sync_copy(src_hbm.at[pl.ds(base, _CHT)], src_v)
    pltpu.sync_copy(dst_hbm.at[pl.ds(base, _CHT)], dst_v)
    pltpu.sync_copy(c_hbm.at[pl.ds(wid * _CHT, _CHT)], c_v)

    def body(i, carry):
        off = i * _L
        si = src_v[pl.ds(off, _L)]
        di = dst_v[pl.ds(off, _L)] + _N
        m = (plsc.load_gather(tab_v, [si]) + plsc.load_gather(tab_v, [di])
             + c_v[pl.ds(off, _L)])
        y_v[pl.ds(off, _L)] = _sigmoid(m)
        return carry

    lax.fori_loop(0, _CHT // _L, body, 0)
    pltpu.sync_copy(y_v, y_hbm.at[pl.ds(wid * _CHT, _CHT)])


@jax.jit
def kernel(node_feats, edge_feats, edge_index, W_src, b_src, W_dst, b_dst,
           W_edge, b_edge):
    src = edge_index[0].astype(jnp.int32)
    dst = edge_index[1].astype(jnp.int32)
    w2 = jnp.concatenate([W_src, W_dst], axis=0)
    b2 = jnp.stack([b_src, b_dst + b_edge]).reshape(2, 1)

    gates = pl.pallas_call(
        _gates_body,
        out_shape=jax.ShapeDtypeStruct((2, _N), jnp.float32),
    )(node_feats, w2, b2)
    table = gates.reshape(2 * _N)

    mesh = plsc.VectorSubcoreMesh(core_axis_name="c", subcore_axis_name="s")
    sc_params = pltpu.CompilerParams(needs_layout_passes=False)

    sc_heavy = pl.kernel(
        _sc_heavy_body,
        out_type=jax.ShapeDtypeStruct((_ESC,), jnp.float32),
        mesh=mesh,
        compiler_params=sc_params,
        scratch_types=[
            pltpu.VMEM((2 * _N,), jnp.float32),
            pltpu.VMEM((_CH,), jnp.int32),
            pltpu.VMEM((_CH,), jnp.int32),
            pltpu.VMEM((_D,), jnp.float32),
            pltpu.VMEM((_CH,), jnp.float32),
            pltpu.VMEM((_G * _D,), jnp.float32),
            pltpu.VMEM((_G * _D,), jnp.float32),
            pltpu.VMEM((_L,), jnp.float32),
            pltpu.SemaphoreType.DMA,
            pltpu.SemaphoreType.DMA,
        ],
    )
    y_sc = sc_heavy(table, src, dst, edge_feats.reshape(_E * _D),
                    W_edge.reshape(_D))

    c = pl.pallas_call(
        _edge_body,
        grid=(_ETC // _BLK,),
        in_specs=[
            pl.BlockSpec((_BLK, _D), lambda i: (i + _ESC // _BLK, 0)),
            pl.BlockSpec((1, _D), lambda i: (0, 0)),
        ],
        out_specs=pl.BlockSpec((_BLK, 1), lambda i: (i, 0)),
        out_shape=jax.ShapeDtypeStruct((_ETC, 1), jnp.float32),
    )(edge_feats, W_edge)

    sc_final = pl.kernel(
        _sc_final_body,
        out_type=jax.ShapeDtypeStruct((_ETC,), jnp.float32),
        mesh=mesh,
        compiler_params=sc_params,
        scratch_types=[
            pltpu.VMEM((2 * _N,), jnp.float32),
            pltpu.VMEM((_CHT,), jnp.int32),
            pltpu.VMEM((_CHT,), jnp.int32),
            pltpu.VMEM((_CHT,), jnp.float32),
            pltpu.VMEM((_CHT,), jnp.float32),
        ],
    )
    y_tc = sc_final(table, src, dst, c.reshape(_ETC))

    return jnp.concatenate([y_sc, y_tc]).reshape(_E, 1)


# speedup vs baseline: 3.1910x; 1.3250x over previous
import functools

import jax
import jax.numpy as jnp
from jax import lax
from jax.experimental import pallas as pl
from jax.experimental.pallas import tpu as pltpu
from jax.experimental.pallas import tpu_sc as plsc

_N = 10000
_E = 320000
_D = 128
_NC = 2
_NS = 16
_NW = _NC * _NS
_L = 16

_BLK = 12800
_ESC = 12 * _BLK
_ETC = _E - _ESC
_CH = _ESC // _NW
_G = 160
_NG = _CH // _G
_CHT = _ETC // _NW


def _gates_body(nf_ref, w2_ref, b2_ref, out_ref):
    out_ref[...] = lax.dot_general(
        w2_ref[...], nf_ref[...],
        (((1,), (1,)), ((), ())),
        preferred_element_type=jnp.float32,
    ) + b2_ref[...]


def _edge_body(ef_ref, we_ref, c_ref):
    c_ref[...] = lax.dot_general(
        ef_ref[...], we_ref[...],
        (((1,), (1,)), ((), ())),
        preferred_element_type=jnp.float32,
    )


def _sigmoid(m):
    return 1.0 / (1.0 + jnp.exp(-m))


def _vperm(v, p):
    return lax.gather(
        v, p.reshape(_L, 1),
        lax.GatherDimensionNumbers(
            offset_dims=(), collapsed_slice_dims=(0,), start_index_map=(0,)),
        slice_sizes=(1,),
        mode=lax.GatherScatterMode.PROMISE_IN_BOUNDS)


def _sc_heavy_body(tab_hbm, src_hbm, dst_hbm, ef_hbm, w_hbm, y_hbm,
                   tab_v, src_v, dst_v, w_v, y_v, tile0, tile1, dot_v,
                   sem0, sem1):
    cid = lax.axis_index("c")
    sid = lax.axis_index("s")
    wid = sid * _NC + cid
    ebase = wid * _CH
    pltpu.sync_copy(tab_hbm, tab_v)
    pltpu.sync_copy(src_hbm.at[pl.ds(ebase, _CH)], src_v)
    pltpu.sync_copy(dst_hbm.at[pl.ds(ebase, _CH)], dst_v)
    pltpu.sync_copy(w_hbm, w_v)

    wvecs = [w_v[pl.ds(cch * _L, _L)] for cch in range(_D // _L)]
    lane = lax.iota(jnp.int32, _L)
    perms = [lane ^ (1 << s) for s in range(4)]
    onehots = [lane == e for e in range(_L)]

    def start(g, tile, sem):
        pltpu.make_async_copy(
            ef_hbm.at[pl.ds((ebase + g * _G) * _D, _G * _D)], tile, sem
        ).start()

    def wait(g, tile, sem):
        pltpu.make_async_copy(
            ef_hbm.at[pl.ds((ebase + g * _G) * _D, _G * _D)], tile, sem
        ).wait()

    def compute(g, tile):
        def jbody(j, carry):
            base_j = j * (_L * _D)
            vs = []
            for e in range(_L):
                row = base_j + e * _D
                parts = [tile[pl.ds(row + cch * _L, _L)] * wvecs[cch]
                         for cch in range(_D // _L)]
                while len(parts) > 1:
                    parts = [parts[i] + parts[i + 1]
                             for i in range(0, len(parts), 2)]
                vs.append(parts[0])
            for p in perms:
                vs = [v + _vperm(v, p) for v in vs]
            for e in range(_L):
                plsc.store_scatter(dot_v, [lane], vs[e], mask=onehots[e])
            e_off = g * _G + j * _L
            si = src_v[pl.ds(e_off, _L)]
            di = dst_v[pl.ds(e_off, _L)] + _N
            m = (dot_v[pl.ds(0, _L)] + plsc.load_gather(tab_v, [si])
                 + plsc.load_gather(tab_v, [di]))
            y_v[pl.ds(e_off, _L)] = _sigmoid(m)
            return carry

        lax.fori_loop(0, _G // _L, jbody, 0)

    start(0, tile0, sem0)

    def body2(t, carry):
        g0 = 2 * t
        wait(g0, tile0, sem0)
        start(g0 + 1, tile1, sem1)
        compute(g0, tile0)
        wait(g0 + 1, tile1, sem1)

        @pl.when(t < _NG // 2 - 1)
        def _():
            start(g0 + 2, tile0, sem0)

        compute(g0 + 1, tile1)
        return carry

    lax.fori_loop(0, _NG // 2, body2, 0)
    pltpu.sync_copy(y_v, y_hbm.at[pl.ds(ebase, _CH)])


def _sc_final_body(tab_hbm, src_hbm, dst_hbm, c_hbm, y_hbm,
                   tab_v, src_v, dst_v, c_v, y_v):
    cid = lax.axis_index("c")
    sid = lax.axis_index("s")
    wid = sid * _NC + cid
    base = _ESC + wid * _CHT
    pltpu.sync_copy(tab_hbm, tab_v)
    pltpu.sync_copy(src_hbm.at[pl.ds(base, _CHT)], src_v)
    pltpu.sync_copy(dst_hbm.at[pl.ds(base, _CHT)], dst_v)
    pltpu.sync_copy(c_hbm.at[pl.ds(wid * _CHT, _CHT)], c_v)

    def body(i, carry):
        off = i * _L
        si = src_v[pl.ds(off, _L)]
        di = dst_v[pl.ds(off, _L)] + _N
        m = (plsc.load_gather(tab_v, [si]) + plsc.load_gather(tab_v, [di])
             + c_v[pl.ds(off, _L)])
        y_v[pl.ds(off, _L)] = _sigmoid(m)
        return carry

    lax.fori_loop(0, _CHT // _L, body, 0)
    pltpu.sync_copy(y_v, y_hbm.at[pl.ds(wid * _CHT, _CHT)])


@jax.jit
def kernel(node_feats, edge_feats, edge_index, W_src, b_src, W_dst, b_dst,
           W_edge, b_edge):
    src = edge_index[0].astype(jnp.int32)
    dst = edge_index[1].astype(jnp.int32)
    w2 = jnp.concatenate([W_src, W_dst], axis=0)
    b2 = jnp.stack([b_src, b_dst + b_edge]).reshape(2, 1)

    gates = pl.pallas_call(
        _gates_body,
        out_shape=jax.ShapeDtypeStruct((2, _N), jnp.float32),
    )(node_feats, w2, b2)
    table = gates.reshape(2 * _N)

    mesh = plsc.VectorSubcoreMesh(core_axis_name="c", subcore_axis_name="s")
    sc_params = pltpu.CompilerParams(needs_layout_passes=False)

    c = pl.pallas_call(
        _edge_body,
        grid=(_ETC // _BLK,),
        in_specs=[
            pl.BlockSpec((_BLK, _D), lambda i: (i + _ESC // _BLK, 0)),
            pl.BlockSpec((1, _D), lambda i: (0, 0)),
        ],
        out_specs=pl.BlockSpec((_BLK, 1), lambda i: (i, 0)),
        out_shape=jax.ShapeDtypeStruct((_ETC, 1), jnp.float32),
    )(edge_feats, W_edge)

    sc_heavy = pl.kernel(
        _sc_heavy_body,
        out_type=jax.ShapeDtypeStruct((_ESC,), jnp.float32),
        mesh=mesh,
        compiler_params=sc_params,
        scratch_types=[
            pltpu.VMEM((2 * _N,), jnp.float32),
            pltpu.VMEM((_CH,), jnp.int32),
            pltpu.VMEM((_CH,), jnp.int32),
            pltpu.VMEM((_D,), jnp.float32),
            pltpu.VMEM((_CH,), jnp.float32),
            pltpu.VMEM((_G * _D,), jnp.float32),
            pltpu.VMEM((_G * _D,), jnp.float32),
            pltpu.VMEM((_L,), jnp.float32),
            pltpu.SemaphoreType.DMA,
            pltpu.SemaphoreType.DMA,
        ],
    )
    y_sc = sc_heavy(table, src, dst, edge_feats.reshape(_E * _D),
                    W_edge.reshape(_D))

    sc_final = pl.kernel(
        _sc_final_body,
        out_type=jax.ShapeDtypeStruct((_ETC,), jnp.float32),
        mesh=mesh,
        compiler_params=sc_params,
        scratch_types=[
            pltpu.VMEM((2 * _N,), jnp.float32),
            pltpu.VMEM((_CHT,), jnp.int32),
            pltpu.VMEM((_CHT,), jnp.int32),
            pltpu.VMEM((_CHT,), jnp.float32),
            pltpu.VMEM((_CHT,), jnp.float32),
        ],
    )
    y_tc = sc_final(table, src, dst, c.reshape(_ETC))

    return jnp.concatenate([y_sc, y_tc]).reshape(_E, 1)


# speedup vs baseline: 3.5093x; 1.0997x over previous
import functools

import jax
import jax.numpy as jnp
from jax import lax
from jax.experimental import pallas as pl
from jax.experimental.pallas import tpu as pltpu
from jax.experimental.pallas import tpu_sc as plsc

_N = 10000
_E = 320000
_D = 128
_NC = 2
_NS = 16
_NW = _NC * _NS
_L = 16

_ESC = _E
_CH = _ESC // _NW
_G = 80
_NG = _CH // _G


def _gates_body(nf_ref, w2_ref, b2_ref, out_ref):
    out_ref[...] = lax.dot_general(
        w2_ref[...], nf_ref[...],
        (((1,), (1,)), ((), ())),
        preferred_element_type=jnp.float32,
    ) + b2_ref[...]


def _edge_body(ef_ref, we_ref, c_ref):
    c_ref[...] = lax.dot_general(
        ef_ref[...], we_ref[...],
        (((1,), (1,)), ((), ())),
        preferred_element_type=jnp.float32,
    )


def _sigmoid(m):
    return 1.0 / (1.0 + jnp.exp(-m))


def _vperm(v, p):
    return lax.gather(
        v, p.reshape(_L, 1),
        lax.GatherDimensionNumbers(
            offset_dims=(), collapsed_slice_dims=(0,), start_index_map=(0,)),
        slice_sizes=(1,),
        mode=lax.GatherScatterMode.PROMISE_IN_BOUNDS)


def _sc_heavy_body(tab_hbm, src_hbm, dst_hbm, ef_hbm, w_hbm, y_hbm,
                   tab_v, src_v, dst_v, w_v, y_v, tile0, tile1, dot_v,
                   sem0, sem1):
    cid = lax.axis_index("c")
    sid = lax.axis_index("s")
    wid = sid * _NC + cid
    ebase = wid * _CH
    pltpu.sync_copy(tab_hbm, tab_v)
    pltpu.sync_copy(src_hbm.at[pl.ds(ebase, _CH)], src_v)
    pltpu.sync_copy(dst_hbm.at[pl.ds(ebase, _CH)], dst_v)
    pltpu.sync_copy(w_hbm, w_v)

    wvecs = [w_v[pl.ds(cch * _L, _L)] for cch in range(_D // _L)]
    lane = lax.iota(jnp.int32, _L)
    perms = [lane ^ (1 << s) for s in range(4)]
    onehots = [lane == e for e in range(_L)]

    def start(g, tile, sem):
        pltpu.make_async_copy(
            ef_hbm.at[pl.ds((ebase + g * _G) * _D, _G * _D)], tile, sem
        ).start()

    def wait(g, tile, sem):
        pltpu.make_async_copy(
            ef_hbm.at[pl.ds((ebase + g * _G) * _D, _G * _D)], tile, sem
        ).wait()

    def compute(g, tile):
        def jbody(j, carry):
            base_j = j * (_L * _D)
            vs = []
            for e in range(_L):
                row = base_j + e * _D
                parts = [tile[pl.ds(row + cch * _L, _L)] * wvecs[cch]
                         for cch in range(_D // _L)]
                while len(parts) > 1:
                    parts = [parts[i] + parts[i + 1]
                             for i in range(0, len(parts), 2)]
                vs.append(parts[0])
            for p in perms:
                vs = [v + _vperm(v, p) for v in vs]
            for e in range(_L):
                plsc.store_scatter(dot_v, [lane], vs[e], mask=onehots[e])
            e_off = g * _G + j * _L
            si = src_v[pl.ds(e_off, _L)]
            di = dst_v[pl.ds(e_off, _L)] + _N
            m = (dot_v[pl.ds(0, _L)] + plsc.load_gather(tab_v, [si])
                 + plsc.load_gather(tab_v, [di]))
            y_v[pl.ds(e_off, _L)] = _sigmoid(m)
            return carry

        lax.fori_loop(0, _G // _L, jbody, 0)

    start(0, tile0, sem0)

    def body2(t, carry):
        g0 = 2 * t
        wait(g0, tile0, sem0)

        @pl.when(g0 + 1 < _NG)
        def _():
            start(g0 + 1, tile1, sem1)

        compute(g0, tile0)

        @pl.when(g0 + 1 < _NG)
        def _():
            wait(g0 + 1, tile1, sem1)

            @pl.when(g0 + 2 < _NG)
            def _():
                start(g0 + 2, tile0, sem0)

            compute(g0 + 1, tile1)
        return carry

    lax.fori_loop(0, (_NG + 1) // 2, body2, 0)
    pltpu.sync_copy(y_v, y_hbm.at[pl.ds(ebase, _CH)])


def _sc_final_body(tab_hbm, src_hbm, dst_hbm, c_hbm, y_hbm,
                   tab_v, src_v, dst_v, c_v, y_v):
    cid = lax.axis_index("c")
    sid = lax.axis_index("s")
    wid = sid * _NC + cid
    base = _ESC + wid * _CHT
    pltpu.sync_copy(tab_hbm, tab_v)
    pltpu.sync_copy(src_hbm.at[pl.ds(base, _CHT)], src_v)
    pltpu.sync_copy(dst_hbm.at[pl.ds(base, _CHT)], dst_v)
    pltpu.sync_copy(c_hbm.at[pl.ds(wid * _CHT, _CHT)], c_v)

    def body(i, carry):
        off = i * _L
        si = src_v[pl.ds(off, _L)]
        di = dst_v[pl.ds(off, _L)] + _N
        m = (plsc.load_gather(tab_v, [si]) + plsc.load_gather(tab_v, [di])
             + c_v[pl.ds(off, _L)])
        y_v[pl.ds(off, _L)] = _sigmoid(m)
        return carry

    lax.fori_loop(0, _CHT // _L, body, 0)
    pltpu.sync_copy(y_v, y_hbm.at[pl.ds(wid * _CHT, _CHT)])


@jax.jit
def kernel(node_feats, edge_feats, edge_index, W_src, b_src, W_dst, b_dst,
           W_edge, b_edge):
    src = edge_index[0].astype(jnp.int32)
    dst = edge_index[1].astype(jnp.int32)
    w2 = jnp.concatenate([W_src, W_dst], axis=0)
    b2 = jnp.stack([b_src, b_dst + b_edge]).reshape(2, 1)

    gates = pl.pallas_call(
        _gates_body,
        out_shape=jax.ShapeDtypeStruct((2, _N), jnp.float32),
    )(node_feats, w2, b2)
    table = gates.reshape(2 * _N)

    mesh = plsc.VectorSubcoreMesh(core_axis_name="c", subcore_axis_name="s")
    sc_params = pltpu.CompilerParams(needs_layout_passes=False)

    sc_heavy = pl.kernel(
        _sc_heavy_body,
        out_type=jax.ShapeDtypeStruct((_ESC,), jnp.float32),
        mesh=mesh,
        compiler_params=sc_params,
        scratch_types=[
            pltpu.VMEM((2 * _N,), jnp.float32),
            pltpu.VMEM((_CH,), jnp.int32),
            pltpu.VMEM((_CH,), jnp.int32),
            pltpu.VMEM((_D,), jnp.float32),
            pltpu.VMEM((_CH,), jnp.float32),
            pltpu.VMEM((_G * _D,), jnp.float32),
            pltpu.VMEM((_G * _D,), jnp.float32),
            pltpu.VMEM((_L,), jnp.float32),
            pltpu.SemaphoreType.DMA,
            pltpu.SemaphoreType.DMA,
        ],
    )
    y_sc = sc_heavy(table, src, dst, edge_feats.reshape(_E * _D),
                    W_edge.reshape(_D))
    return y_sc.reshape(_E, 1)
